# Initial kernel scaffold; baseline (speedup 1.0000x reference)
import jax
import jax.numpy as jnp
from jax.experimental import pallas as pl


def kernel(node_feats, W1, b1, W2, b2):
    raise NotImplementedError("write your pallas kernel here")


# speedup vs baseline: 122.7169x; 122.7169x over previous
import numpy as np
import jax
import jax.numpy as jnp
from jax.experimental import pallas as pl

_N = 24
_CHAIN_STARTS = (0, 2, 6, 10, 14, 19)


def _build_mix_coeffs():
    starts = set(_CHAIN_STARTS)
    has_next = np.array(
        [1.0 if (i + 1 < _N and (i + 1) not in starts) else 0.0 for i in range(_N)],
        dtype=np.float64,
    )
    has_prev = np.array(
        [1.0 if (i > 0 and i not in starts) else 0.0 for i in range(_N)],
        dtype=np.float64,
    )
    deg = 1.0 + has_next + has_prev
    dinv = 1.0 / np.sqrt(deg)
    c0 = dinv * dinv
    cm = np.zeros(_N)
    cp = np.zeros(_N)
    for i in range(_N):
        if has_prev[i]:
            cm[i] = dinv[i] * dinv[i - 1]
        if has_next[i]:
            cp[i] = dinv[i] * dinv[i + 1]
    return (
        c0.astype(np.float32),
        cm.astype(np.float32),
        cp.astype(np.float32),
    )


_C0, _CM, _CP = _build_mix_coeffs()


def _make_fused_kernel(gb, n, cout):
    rows = gb * n

    def body(x_ref, w1_ref, b1_ref, w2_ref, b2_ref, c_ref, o_ref):
        x = x_ref[...].reshape(rows, x_ref.shape[-1])
        kc0 = c_ref[0, :, :]
        kcm = c_ref[1, :, :]
        kcp = c_ref[2, :, :]

        def layer(v, w_ref, b_ref):
            y = jnp.dot(v, w_ref[...], preferred_element_type=jnp.float32)
            mixed = (
                kc0 * y
                + kcm * pltpu_roll(y, 1)
                + kcp * pltpu_roll(y, -1)
            )
            return jax.nn.relu(mixed + b_ref[...])

        h = layer(x, w1_ref, b1_ref)
        h = layer(h, w2_ref, b2_ref)
        pooled = h.reshape(gb, n, cout).sum(axis=1) * (1.0 / n)
        o_ref[...] = pooled

    return body


def pltpu_roll(y, shift):
    return jnp.roll(y, shift, axis=0)


def kernel(node_feats, W1, b1, W2, b2):
    Bc, Nc, C = node_feats.shape
    hid = W1.shape[1]
    out = W2.shape[1]
    gb = 512
    grid = (Bc // gb,)
    body = _make_fused_kernel(gb, Nc, out)
    b1r = b1.reshape(1, hid)
    b2r = b2.reshape(1, out)
    rows = gb * Nc
    coeffs = jnp.asarray(
        np.stack(
            [
                np.tile(_C0, gb).reshape(rows, 1),
                np.tile(_CM, gb).reshape(rows, 1),
                np.tile(_CP, gb).reshape(rows, 1),
            ]
        )
    )
    return pl.pallas_call(
        body,
        grid=grid,
        in_specs=[
            pl.BlockSpec((gb, Nc, C), lambda i: (i, 0, 0)),
            pl.BlockSpec((C, hid), lambda i: (0, 0)),
            pl.BlockSpec((1, hid), lambda i: (0, 0)),
            pl.BlockSpec((hid, out), lambda i: (0, 0)),
            pl.BlockSpec((1, out), lambda i: (0, 0)),
            pl.BlockSpec((3, rows, 1), lambda i: (0, 0, 0)),
        ],
        out_specs=pl.BlockSpec((gb, out), lambda i: (i, 0)),
        out_shape=jax.ShapeDtypeStruct((Bc, out), jnp.float32),
    )(node_feats, W1, b1r, W2, b2r, coeffs)


# speedup vs baseline: 166.3519x; 1.3556x over previous
import numpy as np
import jax
import jax.numpy as jnp
from jax.experimental import pallas as pl

_N = 24
_CHAIN_STARTS = (0, 2, 6, 10, 14, 19)


def _build_mix_coeffs():
    starts = set(_CHAIN_STARTS)
    has_next = np.array(
        [1.0 if (i + 1 < _N and (i + 1) not in starts) else 0.0 for i in range(_N)],
        dtype=np.float64,
    )
    has_prev = np.array(
        [1.0 if (i > 0 and i not in starts) else 0.0 for i in range(_N)],
        dtype=np.float64,
    )
    deg = 1.0 + has_next + has_prev
    dinv = 1.0 / np.sqrt(deg)
    c0 = dinv * dinv
    cm = np.zeros(_N)
    cp = np.zeros(_N)
    for i in range(_N):
        if has_prev[i]:
            cm[i] = dinv[i] * dinv[i - 1]
        if has_next[i]:
            cp[i] = dinv[i] * dinv[i + 1]
    return (
        c0.astype(np.float32),
        cm.astype(np.float32),
        cp.astype(np.float32),
    )


_C0, _CM, _CP = _build_mix_coeffs()


def _make_fused_kernel(gb, n, cout):
    rows = gb * n

    def body(x_ref, w1_ref, b1_ref, w2_ref, b2_ref, c_ref, o_ref):
        x = x_ref[...].reshape(rows, x_ref.shape[-1])
        kc0 = c_ref[0:1, :, :]
        kcm = c_ref[1:2, :, :]
        kcp = c_ref[2:3, :, :]

        def layer(v, w_ref, b_ref):
            y = jnp.dot(v, w_ref[...], preferred_element_type=jnp.float32)
            y3 = y.reshape(gb, n, y.shape[-1])
            mixed = (
                kc0 * y3
                + kcm * jnp.roll(y3, 1, axis=1)
                + kcp * jnp.roll(y3, -1, axis=1)
            )
            return jax.nn.relu(mixed + b_ref[...]).reshape(rows, y.shape[-1])

        h = layer(x, w1_ref, b1_ref)
        h = layer(h, w2_ref, b2_ref)
        pooled = h.reshape(gb, n, cout).sum(axis=1) * (1.0 / n)
        o_ref[...] = pooled

    return body


def pltpu_roll(y, shift):
    return jnp.roll(y, shift, axis=0)


def kernel(node_feats, W1, b1, W2, b2):
    Bc, Nc, C = node_feats.shape
    hid = W1.shape[1]
    out = W2.shape[1]
    gb = 512
    grid = (Bc // gb,)
    body = _make_fused_kernel(gb, Nc, out)
    b1r = b1.reshape(1, hid)
    b2r = b2.reshape(1, out)
    coeffs = jnp.asarray(
        np.stack(
            [
                np.broadcast_to(_C0[:, None], (Nc, out)),
                np.broadcast_to(_CM[:, None], (Nc, out)),
                np.broadcast_to(_CP[:, None], (Nc, out)),
            ]
        )
    )
    return pl.pallas_call(
        body,
        grid=grid,
        in_specs=[
            pl.BlockSpec((gb, Nc, C), lambda i: (i, 0, 0)),
            pl.BlockSpec((C, hid), lambda i: (0, 0)),
            pl.BlockSpec((1, hid), lambda i: (0, 0)),
            pl.BlockSpec((hid, out), lambda i: (0, 0)),
            pl.BlockSpec((1, out), lambda i: (0, 0)),
            pl.BlockSpec((3, Nc, out), lambda i: (0, 0, 0)),
        ],
        out_specs=pl.BlockSpec((gb, out), lambda i: (i, 0)),
        out_shape=jax.ShapeDtypeStruct((Bc, out), jnp.float32),
    )(node_feats, W1, b1r, W2, b2r, coeffs)


# speedup vs baseline: 265.8111x; 1.5979x over previous
import numpy as np
import jax
import jax.numpy as jnp
from jax.experimental import pallas as pl

_N = 24
_G4 = 4
_BD = _N * _G4
_CHAIN_STARTS = (0, 2, 6, 10, 14, 19)


def _build_adj():
    starts = set(_CHAIN_STARTS)
    has_next = np.array(
        [1.0 if (i + 1 < _N and (i + 1) not in starts) else 0.0 for i in range(_N)]
    )
    has_prev = np.array(
        [1.0 if (i > 0 and i not in starts) else 0.0 for i in range(_N)]
    )
    deg = 1.0 + has_next + has_prev
    dinv = 1.0 / np.sqrt(deg)
    A = np.zeros((_N, _N))
    for i in range(_N):
        A[i, i] = dinv[i] * dinv[i]
        if has_prev[i]:
            A[i, i - 1] = dinv[i] * dinv[i - 1]
        if has_next[i]:
            A[i, i + 1] = dinv[i] * dinv[i + 1]
    bd = np.zeros((_BD, _BD), dtype=np.float32)
    for g in range(_G4):
        bd[g * _N:(g + 1) * _N, g * _N:(g + 1) * _N] = A
    return bd


_BDMAT = _build_adj()


def _make_fused_kernel(gb, n, cout):
    rows = gb * n
    ng = rows // _BD

    def body(x_ref, w1_ref, b1_ref, w2_ref, b2_ref, bd_ref, o_ref):
        x = x_ref[...].reshape(rows, x_ref.shape[-1])
        bd = jnp.broadcast_to(bd_ref[...], (ng, _BD, _BD))

        def layer(v, w_ref, b_ref):
            y = jnp.dot(v, w_ref[...], preferred_element_type=jnp.float32)
            yg = y.reshape(ng, _BD, y.shape[-1])
            mixed = jax.lax.dot_general(
                bd, yg,
                dimension_numbers=(((2,), (1,)), ((0,), (0,))),
                preferred_element_type=jnp.float32,
            )
            return jax.nn.relu(
                mixed + b_ref[...]
            ).reshape(rows, y.shape[-1])

        h = layer(x, w1_ref, b1_ref)
        h = layer(h, w2_ref, b2_ref)
        pooled = h.reshape(gb, n, cout).sum(axis=1) * (1.0 / n)
        o_ref[...] = pooled

    return body


def kernel(node_feats, W1, b1, W2, b2):
    Bc, Nc, C = node_feats.shape
    hid = W1.shape[1]
    out = W2.shape[1]
    gb = 512
    grid = (Bc // gb,)
    body = _make_fused_kernel(gb, Nc, out)
    b1r = b1.reshape(1, hid)
    b2r = b2.reshape(1, out)
    bdmat = jnp.asarray(_BDMAT)
    return pl.pallas_call(
        body,
        grid=grid,
        in_specs=[
            pl.BlockSpec((gb, Nc, C), lambda i: (i, 0, 0)),
            pl.BlockSpec((C, hid), lambda i: (0, 0)),
            pl.BlockSpec((1, hid), lambda i: (0, 0)),
            pl.BlockSpec((hid, out), lambda i: (0, 0)),
            pl.BlockSpec((1, out), lambda i: (0, 0)),
            pl.BlockSpec((_BD, _BD), lambda i: (0, 0)),
        ],
        out_specs=pl.BlockSpec((gb, out), lambda i: (i, 0)),
        out_shape=jax.ShapeDtypeStruct((Bc, out), jnp.float32),
    )(node_feats, W1, b1r, W2, b2r, bdmat)


# speedup vs baseline: 273.9583x; 1.0307x over previous
import numpy as np
import jax
import jax.numpy as jnp
from jax.experimental import pallas as pl

_N = 24
_G4 = 4
_BD = _N * _G4
_CHAIN_STARTS = (0, 2, 6, 10, 14, 19)


def _build_adj():
    starts = set(_CHAIN_STARTS)
    has_next = np.array(
        [1.0 if (i + 1 < _N and (i + 1) not in starts) else 0.0 for i in range(_N)]
    )
    has_prev = np.array(
        [1.0 if (i > 0 and i not in starts) else 0.0 for i in range(_N)]
    )
    deg = 1.0 + has_next + has_prev
    dinv = 1.0 / np.sqrt(deg)
    A = np.zeros((_N, _N))
    for i in range(_N):
        A[i, i] = dinv[i] * dinv[i]
        if has_prev[i]:
            A[i, i - 1] = dinv[i] * dinv[i - 1]
        if has_next[i]:
            A[i, i + 1] = dinv[i] * dinv[i + 1]
    bd = np.zeros((_BD, _BD), dtype=np.float32)
    for g in range(_G4):
        bd[g * _N:(g + 1) * _N, g * _N:(g + 1) * _N] = A
    return bd


_BDMAT = _build_adj()


def _make_fused_kernel(gb, n, cout):
    rows = gb * n
    ng = rows // _BD

    def body(x_ref, w1_ref, b1_ref, w2_ref, b2_ref, bd_ref, o_ref):
        x = x_ref[...].reshape(rows, x_ref.shape[-1])
        bd = jnp.broadcast_to(bd_ref[...], (ng, _BD, _BD))

        def layer(v, w_ref, b_ref):
            y = jnp.dot(v, w_ref[...], preferred_element_type=jnp.float32)
            yg = y.reshape(ng, _BD, y.shape[-1])
            mixed = jax.lax.dot_general(
                bd, yg,
                dimension_numbers=(((2,), (1,)), ((0,), (0,))),
                preferred_element_type=jnp.float32,
            )
            return jax.nn.relu(
                mixed + b_ref[...]
            ).reshape(rows, y.shape[-1])

        h = layer(x, w1_ref, b1_ref)
        h = layer(h, w2_ref, b2_ref)
        pooled = h.reshape(gb, n, cout).sum(axis=1) * (1.0 / n)
        o_ref[...] = pooled

    return body


def kernel(node_feats, W1, b1, W2, b2):
    Bc, Nc, C = node_feats.shape
    hid = W1.shape[1]
    out = W2.shape[1]
    gb = 1024
    grid = (Bc // gb,)
    body = _make_fused_kernel(gb, Nc, out)
    b1r = b1.reshape(1, hid)
    b2r = b2.reshape(1, out)
    bdmat = jnp.asarray(_BDMAT)
    return pl.pallas_call(
        body,
        grid=grid,
        in_specs=[
            pl.BlockSpec((gb, Nc, C), lambda i: (i, 0, 0)),
            pl.BlockSpec((C, hid), lambda i: (0, 0)),
            pl.BlockSpec((1, hid), lambda i: (0, 0)),
            pl.BlockSpec((hid, out), lambda i: (0, 0)),
            pl.BlockSpec((1, out), lambda i: (0, 0)),
            pl.BlockSpec((_BD, _BD), lambda i: (0, 0)),
        ],
        out_specs=pl.BlockSpec((gb, out), lambda i: (i, 0)),
        out_shape=jax.ShapeDtypeStruct((Bc, out), jnp.float32),
    )(node_feats, W1, b1r, W2, b2r, bdmat)


# speedup vs baseline: 273.9792x; 1.0001x over previous
import numpy as np
import jax
import jax.numpy as jnp
from jax.experimental import pallas as pl
from jax.experimental.pallas import tpu as pltpu

_N = 24
_G4 = 4
_BD = _N * _G4
_CHAIN_STARTS = (0, 2, 6, 10, 14, 19)


def _build_adj():
    starts = set(_CHAIN_STARTS)
    has_next = np.array(
        [1.0 if (i + 1 < _N and (i + 1) not in starts) else 0.0 for i in range(_N)]
    )
    has_prev = np.array(
        [1.0 if (i > 0 and i not in starts) else 0.0 for i in range(_N)]
    )
    deg = 1.0 + has_next + has_prev
    dinv = 1.0 / np.sqrt(deg)
    A = np.zeros((_N, _N))
    for i in range(_N):
        A[i, i] = dinv[i] * dinv[i]
        if has_prev[i]:
            A[i, i - 1] = dinv[i] * dinv[i - 1]
        if has_next[i]:
            A[i, i + 1] = dinv[i] * dinv[i + 1]
    bd = np.zeros((_BD, _BD), dtype=np.float32)
    for g in range(_G4):
        bd[g * _N:(g + 1) * _N, g * _N:(g + 1) * _N] = A
    return bd


_BDMAT = _build_adj()


def _make_fused_kernel(gb, n, cout):
    rows = gb * n
    ng = rows // _BD

    def body(x_ref, w1_ref, b1_ref, w2_ref, b2_ref, bd_ref, o_ref):
        x = x_ref[...].reshape(rows, x_ref.shape[-1])
        bd = jnp.broadcast_to(bd_ref[...], (ng, _BD, _BD))

        def layer(v, w_ref, b_ref):
            y = jnp.dot(v, w_ref[...], preferred_element_type=jnp.float32)
            yg = y.reshape(ng, _BD, y.shape[-1])
            mixed = jax.lax.dot_general(
                bd, yg,
                dimension_numbers=(((2,), (1,)), ((0,), (0,))),
                preferred_element_type=jnp.float32,
            )
            return jax.nn.relu(
                mixed + b_ref[...]
            ).reshape(rows, y.shape[-1])

        h = layer(x, w1_ref, b1_ref)
        h = layer(h, w2_ref, b2_ref)
        pooled = h.reshape(gb, n, cout).sum(axis=1) * (1.0 / n)
        o_ref[...] = pooled

    return body


def kernel(node_feats, W1, b1, W2, b2):
    Bc, Nc, C = node_feats.shape
    hid = W1.shape[1]
    out = W2.shape[1]
    gb = 1024
    grid = (Bc // gb,)
    body = _make_fused_kernel(gb, Nc, out)
    b1r = b1.reshape(1, hid)
    b2r = b2.reshape(1, out)
    bdmat = jnp.asarray(_BDMAT)
    return pl.pallas_call(
        body,
        grid=grid,
        compiler_params=pltpu.CompilerParams(
            dimension_semantics=("parallel",),
        ),
        in_specs=[
            pl.BlockSpec((gb, Nc, C), lambda i: (i, 0, 0)),
            pl.BlockSpec((C, hid), lambda i: (0, 0)),
            pl.BlockSpec((1, hid), lambda i: (0, 0)),
            pl.BlockSpec((hid, out), lambda i: (0, 0)),
            pl.BlockSpec((1, out), lambda i: (0, 0)),
            pl.BlockSpec((_BD, _BD), lambda i: (0, 0)),
        ],
        out_specs=pl.BlockSpec((gb, out), lambda i: (i, 0)),
        out_shape=jax.ShapeDtypeStruct((Bc, out), jnp.float32),
    )(node_feats, W1, b1r, W2, b2r, bdmat)
